# baseline (speedup 1.0000x reference)
import functools

import jax
import jax.numpy as jnp
from jax import lax
from jax.experimental import pallas as pl
from jax.experimental.pallas import tpu as pltpu
from jax.experimental.pallas import tpu_sc as plsc

HIDDEN = 64
SCALE = 8.0
LANES = 16
NUM_CORES = 2
NUM_SUBCORES = 16
NW = NUM_CORES * NUM_SUBCORES
TOTAL = 4096 * 200
PER_W = TOTAL // NW
CHUNK = 512
SUB = 128
NSUB = CHUNK // SUB
STEPS = PER_W // CHUNK

_mesh = plsc.VectorSubcoreMesh(core_axis_name="c", subcore_axis_name="s")


@functools.partial(
    pl.kernel,
    out_type=jax.ShapeDtypeStruct((TOTAL, HIDDEN), jnp.float32),
    mesh=_mesh,
    scratch_types=[
        pltpu.VMEM((CHUNK,), jnp.int32),
        pltpu.VMEM((CHUNK, HIDDEN), jnp.float32),
        pltpu.SemaphoreType.DMA,
    ],
    compiler_params=pltpu.CompilerParams(use_tc_tiling_on_sc=False),
)
def _embed(table_hbm, idx_hbm, out_hbm, idx_v, rows_v, sem):
    wid = lax.axis_index("s") * NUM_CORES + lax.axis_index("c")
    wbase = wid * PER_W

    def step(i, carry):
        base = wbase + i * CHUNK
        pltpu.sync_copy(idx_hbm.at[pl.ds(base, CHUNK)], idx_v)
        copies = [
            pltpu.async_copy(
                table_hbm.at[idx_v.at[pl.ds(j * SUB, SUB)]],
                rows_v.at[pl.ds(j * SUB, SUB)],
                sem,
            )
            for j in range(NSUB)
        ]
        for c in copies:
            c.wait()

        def group(g, rc):
            iv = idx_v[pl.ds(g * LANES, LANES)]
            sv = jnp.where(iv != 0, jnp.float32(SCALE), jnp.float32(0.0))
            for j in range(LANES):
                r = g * LANES + j
                scale = sv[j]
                for cb in range(HIDDEN // LANES):
                    seg = rows_v[r, pl.ds(cb * LANES, LANES)]
                    rows_v[r, pl.ds(cb * LANES, LANES)] = seg * scale
            return rc

        lax.fori_loop(0, CHUNK // LANES, group, 0)
        pltpu.sync_copy(rows_v, out_hbm.at[pl.ds(base, CHUNK)])
        return carry

    lax.fori_loop(0, STEPS, step, 0)


def kernel(x, shared_weights):
    xf = x.reshape(-1).astype(jnp.int32)
    out = _embed(shared_weights, xf)
    return out.reshape(x.shape[0], x.shape[1], HIDDEN)


# speedup vs baseline: 1.0376x; 1.0376x over previous
import functools

import jax
import jax.numpy as jnp
from jax import lax
from jax.experimental import pallas as pl
from jax.experimental.pallas import tpu as pltpu
from jax.experimental.pallas import tpu_sc as plsc

HIDDEN = 64
SCALE = 8.0
LANES = 16
NUM_CORES = 2
NUM_SUBCORES = 16
NW = NUM_CORES * NUM_SUBCORES
XROWS = 4096
XCOLS = 200
TOTAL = XROWS * XCOLS
ROWS_PER_W = XROWS // NW
R = 2
CHUNK = R * XCOLS
STEPS = ROWS_PER_W // R
UNITS = 500000
SUBS = [(0, 128), (128, 128), (256, 128), (384, 16)]
GROUP_OFFS = [min(16 * g, XCOLS - LANES) for g in range((XCOLS + LANES - 1) // LANES)]

_mesh = plsc.VectorSubcoreMesh(core_axis_name="c", subcore_axis_name="s")


@functools.partial(
    pl.kernel,
    out_type=jax.ShapeDtypeStruct((TOTAL, HIDDEN), jnp.float32),
    mesh=_mesh,
    scratch_types=[
        pltpu.VMEM((R, XCOLS), jnp.int32),
        pltpu.VMEM((CHUNK,), jnp.int32),
        pltpu.VMEM((CHUNK,), jnp.float32),
        pltpu.VMEM((CHUNK,), jnp.int32),
        pltpu.VMEM((CHUNK, 2 * HIDDEN), jnp.float32),
        pltpu.VMEM((CHUNK, HIDDEN), jnp.float32),
        pltpu.SemaphoreType.DMA,
    ],
    compiler_params=pltpu.CompilerParams(use_tc_tiling_on_sc=True),
)
def _embed(x_hbm, tab_hbm, out_hbm, idx_v, uidx_v, sv_v, pv_v, unit_v, rows_v, sem):
    wid = lax.axis_index("s") * NUM_CORES + lax.axis_index("c")
    row0 = wid * ROWS_PER_W

    def step(i, carry):
        xr = row0 + i * R
        base = xr * XCOLS
        pltpu.sync_copy(x_hbm.at[pl.ds(xr, R)], idx_v)
        for j in range(R):
            for off in GROUP_OFFS:
                iv = idx_v[j, pl.ds(off, LANES)]
                fo = j * XCOLS + off
                uidx_v[pl.ds(fo, LANES)] = lax.shift_right_logical(iv, 1)
                pv_v[pl.ds(fo, LANES)] = lax.shift_left(
                    jnp.bitwise_and(iv, 1), 6
                )
                sv_v[pl.ds(fo, LANES)] = jnp.where(
                    iv != 0, jnp.float32(SCALE), jnp.float32(0.0)
                )
        copies = [
            pltpu.async_copy(
                tab_hbm.at[uidx_v.at[pl.ds(o, l)]],
                unit_v.at[pl.ds(o, l)],
                sem,
            )
            for (o, l) in SUBS
        ]
        for c in copies:
            c.wait()

        def group(g, gc):
            sv_vec = sv_v[pl.ds(g * LANES, LANES)]
            pv_vec = pv_v[pl.ds(g * LANES, LANES)]
            for j in range(LANES):
                r = g * LANES + j
                scale = sv_vec[j]
                off = pv_vec[j]
                for c in range(HIDDEN // LANES):
                    seg = unit_v[r, pl.ds(off + c * LANES, LANES)]
                    rows_v[r, pl.ds(c * LANES, LANES)] = seg * scale
            return gc

        lax.fori_loop(0, CHUNK // LANES, group, 0)
        pltpu.sync_copy(rows_v, out_hbm.at[pl.ds(base, CHUNK)])
        return carry

    lax.fori_loop(0, STEPS, step, 0)


def kernel(x, shared_weights):
    xi = x.astype(jnp.int32)
    tab = shared_weights.reshape(UNITS, 2 * HIDDEN)
    out = _embed(xi, tab)
    return out.reshape(XROWS, XCOLS, HIDDEN)
